# baseline (speedup 1.0000x reference)
import functools

import jax
import jax.numpy as jnp
from jax import lax
from jax.experimental import pallas as pl
from jax.experimental.pallas import tpu as pltpu
from jax.experimental.pallas import tpu_sc as plsc

_NUM_INSTANCE = 16384
_FEAT_LEN = 768
_POS_NUM = 8192
_NC, _NS = 2, 16
_NW = _NC * _NS
_ROWS_PER_W = _NUM_INSTANCE // _NW


def _queue_update(vis_feat, lag_feat, vis_q, lag_q):
    mesh = plsc.VectorSubcoreMesh(core_axis_name="c", subcore_axis_name="s")
    out_sds = jax.ShapeDtypeStruct((_NUM_INSTANCE, _FEAT_LEN), jnp.float32)

    @functools.partial(
        pl.kernel,
        out_type=(out_sds, out_sds),
        mesh=mesh,
    )
    def body(vis_feat_hbm, lag_feat_hbm, vis_q_hbm, lag_q_hbm,
             vis_out_hbm, lag_out_hbm):
        wid = lax.axis_index("c") * _NS + lax.axis_index("s")
        base = wid * _ROWS_PER_W

        @pl.when(base < _POS_NUM)
        def _copy_feat():
            pltpu.sync_copy(vis_feat_hbm.at[pl.ds(base, _ROWS_PER_W)],
                            vis_out_hbm.at[pl.ds(base, _ROWS_PER_W)])
            pltpu.sync_copy(lag_feat_hbm.at[pl.ds(base, _ROWS_PER_W)],
                            lag_out_hbm.at[pl.ds(base, _ROWS_PER_W)])

        @pl.when(base >= _POS_NUM)
        def _copy_tail():
            pltpu.sync_copy(vis_q_hbm.at[pl.ds(base, _ROWS_PER_W)],
                            vis_out_hbm.at[pl.ds(base, _ROWS_PER_W)])
            pltpu.sync_copy(lag_q_hbm.at[pl.ds(base, _ROWS_PER_W)],
                            lag_out_hbm.at[pl.ds(base, _ROWS_PER_W)])

    return body(vis_feat, lag_feat, vis_q, lag_q)


def kernel(vis_feat, lag_feat, vis_memory_queue, lag_memory_queue):
    return _queue_update(vis_feat, lag_feat, vis_memory_queue,
                         lag_memory_queue)


# speedup vs baseline: 33.9721x; 33.9721x over previous
import functools

import jax
import jax.numpy as jnp
from jax import lax
from jax.experimental import pallas as pl
from jax.experimental.pallas import tpu as pltpu
from jax.experimental.pallas import tpu_sc as plsc

_NUM_INSTANCE = 16384
_FEAT_LEN = 768
_POS_NUM = 8192
_NC, _NS = 2, 16
_NW = _NC * _NS
_ROWS_PER_W = _NUM_INSTANCE // _NW
_CHUNK = 64
_NCHUNK = _ROWS_PER_W // _CHUNK


def _queue_update(vis_feat, lag_feat, vis_q, lag_q):
    mesh = plsc.VectorSubcoreMesh(core_axis_name="c", subcore_axis_name="s")
    out_sds = jax.ShapeDtypeStruct((_NUM_INSTANCE, _FEAT_LEN), jnp.float32)

    @functools.partial(
        pl.kernel,
        out_type=(out_sds, out_sds),
        mesh=mesh,
        scratch_types=[
            pltpu.VMEM((_CHUNK, _FEAT_LEN), jnp.float32),
            pltpu.VMEM((_CHUNK, _FEAT_LEN), jnp.float32),
            pltpu.SemaphoreType.DMA,
            pltpu.SemaphoreType.DMA,
            pltpu.SemaphoreType.DMA,
            pltpu.SemaphoreType.DMA,
        ],
    )
    def body(vis_feat_hbm, lag_feat_hbm, vis_q_hbm, lag_q_hbm,
             vis_out_hbm, lag_out_hbm,
             buf0, buf1, in_sem0, in_sem1, out_sem0, out_sem1):
        wid = lax.axis_index("c") * _NS + lax.axis_index("s")
        base = wid * _ROWS_PER_W
        bufs = (buf0, buf1)
        in_sems = (in_sem0, in_sem1)
        out_sems = (out_sem0, out_sem1)

        def staged_copy(src, dst):
            outs = [None] * _NCHUNK
            for c in range(_NCHUNK):
                b = c % 2
                off = base + c * _CHUNK
                if c >= 2:
                    outs[c - 2].wait()
                inc = pltpu.make_async_copy(
                    src.at[pl.ds(off, _CHUNK)], bufs[b], in_sems[b])
                inc.start()
                inc.wait()
                outs[c] = pltpu.make_async_copy(
                    bufs[b], dst.at[pl.ds(off, _CHUNK)], out_sems[b])
                outs[c].start()
            outs[-2].wait()
            outs[-1].wait()

        @pl.when(base < _POS_NUM)
        def _copy_feat():
            staged_copy(vis_feat_hbm, vis_out_hbm)
            staged_copy(lag_feat_hbm, lag_out_hbm)

        @pl.when(base >= _POS_NUM)
        def _copy_tail():
            staged_copy(vis_q_hbm, vis_out_hbm)
            staged_copy(lag_q_hbm, lag_out_hbm)

    return body(vis_feat, lag_feat, vis_q, lag_q)


def kernel(vis_feat, lag_feat, vis_memory_queue, lag_memory_queue):
    return _queue_update(vis_feat, lag_feat, vis_memory_queue,
                         lag_memory_queue)


# speedup vs baseline: 34.1351x; 1.0048x over previous
import functools

import jax
import jax.numpy as jnp
from jax import lax
from jax.experimental import pallas as pl
from jax.experimental.pallas import tpu as pltpu
from jax.experimental.pallas import tpu_sc as plsc

_NUM_INSTANCE = 16384
_FEAT_LEN = 768
_POS_NUM = 8192
_NC, _NS = 2, 16
_NW = _NC * _NS
_ROWS_PER_W = _NUM_INSTANCE // _NW
_CHUNK = 64
_NCHUNK = _ROWS_PER_W // _CHUNK


def _queue_update(vis_feat, lag_feat, vis_q, lag_q):
    mesh = plsc.VectorSubcoreMesh(core_axis_name="c", subcore_axis_name="s")
    out_sds = jax.ShapeDtypeStruct((_NUM_INSTANCE, _FEAT_LEN), jnp.float32)

    @functools.partial(
        pl.kernel,
        out_type=(out_sds, out_sds),
        mesh=mesh,
        scratch_types=[
            pltpu.VMEM((_CHUNK, _FEAT_LEN), jnp.float32),
            pltpu.VMEM((_CHUNK, _FEAT_LEN), jnp.float32),
            pltpu.SemaphoreType.DMA,
            pltpu.SemaphoreType.DMA,
            pltpu.SemaphoreType.DMA,
            pltpu.SemaphoreType.DMA,
        ],
    )
    def body(vis_feat_hbm, lag_feat_hbm, vis_q_hbm, lag_q_hbm,
             vis_out_hbm, lag_out_hbm,
             buf0, buf1, in_sem0, in_sem1, out_sem0, out_sem1):
        wid = lax.axis_index("c") * _NS + lax.axis_index("s")
        base = wid * _ROWS_PER_W
        bufs = (buf0, buf1)
        in_sems = (in_sem0, in_sem1)
        out_sems = (out_sem0, out_sem1)

        def run_pipeline(jobs):
            njobs = len(jobs)
            ins = [None] * njobs
            outs = [None] * njobs

            def start_in(c):
                src, _, off = jobs[c]
                ins[c] = pltpu.make_async_copy(
                    src.at[pl.ds(off, _CHUNK)], bufs[c % 2], in_sems[c % 2])
                ins[c].start()

            start_in(0)
            start_in(1)
            for c in range(njobs):
                b = c % 2
                _, dst, off = jobs[c]
                ins[c].wait()
                outs[c] = pltpu.make_async_copy(
                    bufs[b], dst.at[pl.ds(off, _CHUNK)], out_sems[b])
                outs[c].start()
                if c + 2 < njobs:
                    outs[c].wait()
                    start_in(c + 2)
            outs[-2].wait()
            outs[-1].wait()

        def jobs_for(src_v, src_l):
            jobs = []
            for c in range(_NCHUNK):
                jobs.append((src_v, vis_out_hbm, base + c * _CHUNK))
                jobs.append((src_l, lag_out_hbm, base + c * _CHUNK))
            return jobs

        @pl.when(base < _POS_NUM)
        def _copy_feat():
            run_pipeline(jobs_for(vis_feat_hbm, lag_feat_hbm))

        @pl.when(base >= _POS_NUM)
        def _copy_tail():
            run_pipeline(jobs_for(vis_q_hbm, lag_q_hbm))

    return body(vis_feat, lag_feat, vis_q, lag_q)


def kernel(vis_feat, lag_feat, vis_memory_queue, lag_memory_queue):
    return _queue_update(vis_feat, lag_feat, vis_memory_queue,
                         lag_memory_queue)


# speedup vs baseline: 36.6506x; 1.0737x over previous
import functools

import jax
import jax.numpy as jnp
from jax import lax
from jax.experimental import pallas as pl
from jax.experimental.pallas import tpu as pltpu
from jax.experimental.pallas import tpu_sc as plsc

_NUM_INSTANCE = 16384
_FEAT_LEN = 768
_POS_NUM = 8192
_NC, _NS = 2, 16
_NW = _NC * _NS
_ROWS_PER_W = _NUM_INSTANCE // _NW
_CHUNK = 64
_NCHUNK = _ROWS_PER_W // _CHUNK

_TC_BLOCK = 512
_TC_NBLK = _NUM_INSTANCE // _TC_BLOCK
_TC_FEAT_BLKS = _POS_NUM // _TC_BLOCK


def _sc_queue_update(feat, q):
    mesh = plsc.VectorSubcoreMesh(core_axis_name="c", subcore_axis_name="s")
    out_sds = jax.ShapeDtypeStruct((_NUM_INSTANCE, _FEAT_LEN), jnp.float32)

    @functools.partial(
        pl.kernel,
        out_type=out_sds,
        mesh=mesh,
        scratch_types=[
            pltpu.VMEM((_CHUNK, _FEAT_LEN), jnp.float32),
            pltpu.VMEM((_CHUNK, _FEAT_LEN), jnp.float32),
            pltpu.SemaphoreType.DMA,
            pltpu.SemaphoreType.DMA,
            pltpu.SemaphoreType.DMA,
            pltpu.SemaphoreType.DMA,
        ],
    )
    def body(feat_hbm, q_hbm, out_hbm,
             buf0, buf1, in_sem0, in_sem1, out_sem0, out_sem1):
        wid = lax.axis_index("c") * _NS + lax.axis_index("s")
        base = wid * _ROWS_PER_W
        bufs = (buf0, buf1)
        in_sems = (in_sem0, in_sem1)
        out_sems = (out_sem0, out_sem1)

        def run_pipeline(src):
            ins = [None] * _NCHUNK
            outs = [None] * _NCHUNK

            def start_in(c):
                ins[c] = pltpu.make_async_copy(
                    src.at[pl.ds(base + c * _CHUNK, _CHUNK)],
                    bufs[c % 2], in_sems[c % 2])
                ins[c].start()

            start_in(0)
            start_in(1)
            for c in range(_NCHUNK):
                b = c % 2
                ins[c].wait()
                outs[c] = pltpu.make_async_copy(
                    bufs[b], out_hbm.at[pl.ds(base + c * _CHUNK, _CHUNK)],
                    out_sems[b])
                outs[c].start()
                if c + 2 < _NCHUNK:
                    outs[c].wait()
                    start_in(c + 2)
            outs[-2].wait()
            outs[-1].wait()

        @pl.when(base < _POS_NUM)
        def _copy_feat():
            run_pipeline(feat_hbm)

        @pl.when(base >= _POS_NUM)
        def _copy_tail():
            run_pipeline(q_hbm)

    return body(feat, q)


def _tc_copy_body(feat_ref, q_ref, out_ref):
    i = pl.program_id(0)

    @pl.when(i < _TC_FEAT_BLKS)
    def _():
        out_ref[...] = feat_ref[...]

    @pl.when(i >= _TC_FEAT_BLKS)
    def _():
        out_ref[...] = q_ref[...]


def _tc_queue_update(feat, q):
    return pl.pallas_call(
        _tc_copy_body,
        grid=(_TC_NBLK,),
        in_specs=[
            pl.BlockSpec(
                (_TC_BLOCK, _FEAT_LEN),
                lambda i: (jnp.minimum(i, _TC_FEAT_BLKS - 1), 0)),
            pl.BlockSpec(
                (_TC_BLOCK, _FEAT_LEN),
                lambda i: (jnp.maximum(i, _TC_FEAT_BLKS), 0)),
        ],
        out_specs=pl.BlockSpec((_TC_BLOCK, _FEAT_LEN), lambda i: (i, 0)),
        out_shape=jax.ShapeDtypeStruct((_NUM_INSTANCE, _FEAT_LEN),
                                       jnp.float32),
    )(feat, q)


def kernel(vis_feat, lag_feat, vis_memory_queue, lag_memory_queue):
    new_vis = _sc_queue_update(vis_feat, vis_memory_queue)
    new_lag = _tc_queue_update(lag_feat, lag_memory_queue)
    return (new_vis, new_lag)
